# baseline (speedup 1.0000x reference)
import functools

import jax
import jax.numpy as jnp
from jax import lax
from jax.experimental import pallas as pl
from jax.experimental.pallas import tpu as pltpu
from jax.experimental.pallas import tpu_sc as plsc

HID = 64
LANES = 16
NC, NS = 2, 16
NW = NC * NS
CHUNK = 128
NSET = 3


BLK = 2000


def _tables_body(x_ref, y_ref, w_ref, b_ref, txy_ref, ts_ref):
    wx = w_ref[:, :HID]
    wy = w_ref[:, HID:2 * HID]
    ws = w_ref[:, 2 * HID:2 * HID + 1]
    dims = (((1,), (1,)), ((), ()))
    xp = (
        lax.dot_general(x_ref[...], wx, dims, preferred_element_type=jnp.float32)
        + b_ref[...]
    )
    yp = lax.dot_general(y_ref[...], wy, dims, preferred_element_type=jnp.float32)
    txy_ref[...] = jnp.concatenate([xp, yp], axis=1)
    rows = (
        lax.broadcasted_iota(jnp.int32, (BLK, 1), 0) + pl.program_id(0) * BLK
    ).astype(jnp.float32)
    dims = (((1,), (1,)), ((), ()))
    sp = lax.dot_general(rows, ws, dims, preferred_element_type=jnp.float32)
    ts_ref[...] = jnp.concatenate([sp, sp], axis=1)


def _project_tables(x_table, y_table, W, b):
    V = x_table.shape[0]
    assert V % BLK == 0
    tspec = pl.BlockSpec((BLK, HID), lambda i: (i, 0))
    ospec = pl.BlockSpec((BLK, 2 * HID), lambda i: (i, 0))
    return pl.pallas_call(
        _tables_body,
        grid=(V // BLK,),
        in_specs=[
            tspec,
            tspec,
            pl.BlockSpec((HID, 2 * HID + 1), lambda i: (0, 0)),
            pl.BlockSpec((1, HID), lambda i: (0, 0)),
        ],
        out_specs=[ospec, ospec],
        out_shape=[jax.ShapeDtypeStruct((V, 2 * HID), jnp.float32)] * 2,
    )(x_table, y_table, W, b.reshape(1, HID))


SUB = 128
KSUB = CHUNK // SUB


def _make_sc_gather(N):
    rows_per_w = N // NW
    nchunk = rows_per_w // CHUNK
    nblk_w = rows_per_w // SUB
    assert rows_per_w % CHUNK == 0 and nchunk % 2 == 0 and nchunk >= 4

    mesh = plsc.VectorSubcoreMesh(core_axis_name="c", subcore_axis_name="s")

    idx_t = pltpu.VMEM((KSUB, SUB), jnp.int32)
    buf_t = pltpu.VMEM((CHUNK, HID), jnp.float32)

    @functools.partial(
        pl.kernel,
        out_type=jax.ShapeDtypeStruct((N, HID), jnp.float32),
        mesh=mesh,
        scratch_types=[idx_t] * (3 * NSET) + [buf_t] * (3 * NSET)
        + [pltpu.SemaphoreType.DMA] * (3 * NSET),
        compiler_params=pltpu.CompilerParams(use_tc_tiling_on_sc=False),
    )
    def sc_gather(idx_hbm, txy_hbm, ts_hbm, out_hbm, *scratch):
        ib, bb, sb = 0, 3 * NSET, 6 * NSET
        sets = tuple(
            (
                scratch[ib + 3 * k : ib + 3 * k + 3],
                scratch[bb + 3 * k : bb + 3 * k + 3],
                scratch[sb + k],
                scratch[sb + NSET + k],
                scratch[sb + 2 * NSET + k],
            )
            for k in range(NSET)
        )
        tables = (txy_hbm, txy_hbm, ts_hbm)

        wid = lax.axis_index("s") * NC + lax.axis_index("c")
        row_base = wid * rows_per_w
        blk_base = wid * nblk_w

        def issue_idx(s, c):
            idx, _, isem, _, _ = s
            blk = blk_base + c * KSUB
            for t in range(3):
                pltpu.async_copy(idx_hbm.at[t, pl.ds(blk, KSUB)], idx[t], isem)

        def wait_idx(s):
            idx, _, isem, _, _ = s
            for t in range(3):
                pltpu.make_async_copy(
                    idx_hbm.at[t, pl.ds(0, KSUB)], idx[t], isem
                ).wait()

        def issue_gather(s, c):
            idx, buf, _, gsem, _ = s
            for t in range(3):
                for j in range(KSUB):
                    pltpu.async_copy(
                        tables[t].at[idx[t].at[j]],
                        buf[t].at[pl.ds(j * SUB, SUB)],
                        gsem,
                    )

        def wait_gather(s):
            _, buf, _, gsem, _ = s
            for t in range(3):
                pltpu.make_async_copy(
                    out_hbm.at[pl.ds(0, CHUNK)], buf[t], gsem
                ).wait()

        def issue_write(s, c):
            _, buf, _, _, wsem = s
            off = row_base + c * CHUNK
            pltpu.async_copy(buf[0], out_hbm.at[pl.ds(off, CHUNK)], wsem)

        def wait_write(s):
            _, buf, _, _, wsem = s
            pltpu.make_async_copy(
                buf[0], out_hbm.at[pl.ds(0, CHUNK)], wsem
            ).wait()

        def combine(s):
            _, buf, _, _, _ = s
            bx, by, bs = buf

            def row_body(i, c):
                for j in range(HID // LANES):
                    sl = pl.ds(j * LANES, LANES)
                    bx[i, sl] = bx[i, sl] + by[i, sl] + bs[i, sl]
                return c

            lax.fori_loop(0, CHUNK, row_body, 0, unroll=2)

        for k in range(NSET):
            issue_idx(sets[k], k)
        for k in range(NSET - 1):
            wait_idx(sets[k])
            issue_gather(sets[k], k)

        niter = (nchunk + NSET - 1) // NSET

        def outer(i, carry):
            g = i * NSET
            for b in range(NSET):
                s = sets[b]
                sn = sets[(b + NSET - 1) % NSET]
                c = g + b

                @pl.when(c < nchunk)
                def _():
                    wait_gather(s)

                    @pl.when(c + NSET < nchunk)
                    def _():
                        issue_idx(s, c + NSET)

                    @pl.when(c + NSET - 1 < nchunk)
                    def _():
                        wait_idx(sn)

                        @pl.when(c >= 1)
                        def _():
                            wait_write(sn)

                        issue_gather(sn, c + NSET - 1)

                    combine(s)
                    issue_write(s, c)

            return carry

        lax.fori_loop(0, niter, outer, 0)
        for k in range(NSET):
            wait_write(sets[k])

    return sc_gather


def kernel(src, x_table, y_table, W, b):
    B, L, _ = src.shape
    N = B * L
    V = x_table.shape[0]
    half = jnp.array([0, 1, 0], jnp.int32)[:, None, None]
    idx = (
        jnp.transpose(src, (2, 0, 1)).astype(jnp.int32) * 2 + half
    ).reshape(3, N // SUB, SUB)
    txy, ts = _project_tables(x_table, y_table, W, b)
    out = _make_sc_gather(N)(idx, txy.reshape(2 * V, HID), ts.reshape(2 * V, HID))
    return out.reshape(B, L, HID)
